# baseline (speedup 1.0000x reference)
import functools

import jax
import jax.numpy as jnp
from jax import lax
from jax.experimental import pallas as pl
from jax.experimental.pallas import tpu as pltpu
from jax.experimental.pallas import tpu_sc as plsc

B = 32
D = 512
N = 1024
D_OUT = 2048
NW = 32
DPW = D_OUT // NW
RC = 16
NV = N // 16


def _mm_body(x_ref, a_ref, o_ref):
    xb = x_ref[0]
    a = a_ref[...]
    o_ref[0] = lax.dot_general(
        a, xb, (((0,), (0,)), ((), ())),
        preferred_element_type=jnp.float32,
    )


def _tc_matmul(input, A):
    TQ = 256
    return pl.pallas_call(
        _mm_body,
        grid=(B, D_OUT // TQ),
        in_specs=[
            pl.BlockSpec((1, D, N), lambda b, t: (b, 0, 0)),
            pl.BlockSpec((D, TQ), lambda b, t: (0, t)),
        ],
        out_specs=pl.BlockSpec((1, TQ, N), lambda b, t: (b, t, 0)),
        out_shape=jax.ShapeDtypeStruct((B, D_OUT, N), jnp.float32),
    )(input, A)


def _sig(K, v):
    if K > 1024:
        return False
    return bool(v & (K // 16))


def _sort_row(rows_v, j):
    for v in range(NV):
        s = rows_v[j, pl.ds(16 * v, 16)]
        if _sig(16, v):
            s = -s
        s = plsc.sort_key_val(s, s)[0]
        if _sig(16, v) != _sig(32, v):
            s = -s
        rows_v[j, pl.ds(16 * v, 16)] = s
    for K in (32, 64, 128, 256, 512, 1024):
        d = K // 32
        while d >= 1:
            for a in range(NV):
                if a & d:
                    continue
                x = rows_v[j, pl.ds(16 * a, 16)]
                y = rows_v[j, pl.ds(16 * (a + d), 16)]
                mn = jnp.minimum(x, y)
                mx = jnp.maximum(x, y)
                if d == 1:
                    mn = plsc.sort_key_val(mn, mn)[0]
                    mx = plsc.sort_key_val(mx, mx)[0]
                    if _sig(K, a) != _sig(2 * K, a):
                        mn = -mn
                    if _sig(K, a + 1) != _sig(2 * K, a + 1):
                        mx = -mx
                rows_v[j, pl.ds(16 * a, 16)] = mn
                rows_v[j, pl.ds(16 * (a + d), 16)] = mx
            d //= 2


def _sc_sort_dot(prod, wt):
    mesh = plsc.VectorSubcoreMesh(core_axis_name="c", subcore_axis_name="s")

    @functools.partial(
        pl.kernel,
        mesh=mesh,
        compiler_params=pltpu.CompilerParams(needs_layout_passes=False),
        out_type=jax.ShapeDtypeStruct((B, D_OUT), jnp.float32),
        scratch_types=[
            pltpu.VMEM((DPW, N), jnp.float32),
            pltpu.VMEM((RC, N), jnp.float32),
            pltpu.VMEM((B, DPW), jnp.float32),
        ],
    )
    def k(prod_hbm, wt_hbm, out_hbm, wt_v, rows_v, out_v):
        wid = lax.axis_index("s") * 2 + lax.axis_index("c")
        d0 = wid * DPW
        pltpu.sync_copy(wt_hbm.at[pl.ds(d0, DPW)], wt_v)

        lane = lax.broadcasted_iota(jnp.int32, (16,), 0)

        def chunk_body(m, carry):
            b = m // (DPW // RC)
            sc = m % (DPW // RC)
            pltpu.sync_copy(prod_hbm.at[b, pl.ds(d0 + sc * RC, RC)], rows_v)

            def row_body(j, curr):
                _sort_row(rows_v, j)
                acc = jnp.zeros((16,), jnp.float32)
                for v in range(NV):
                    acc = acc + (rows_v[j, pl.ds(16 * v, 16)]
                                 * wt_v[sc * RC + j, pl.ds(16 * v, 16)])
                tot = jnp.sum(acc)
                return jnp.where(lane == j, tot, curr)

            curr = lax.fori_loop(0, RC, row_body, jnp.zeros((16,), jnp.float32))
            out_v[b, pl.ds(sc * RC, 16)] = curr
            return carry

        lax.fori_loop(0, B * (DPW // RC), chunk_body, 0)

        def out_body(b, carry):
            pltpu.sync_copy(out_v.at[b], out_hbm.at[b, pl.ds(d0, DPW)])
            return carry

        lax.fori_loop(0, B, out_body, 0)

    return k(prod, wt)


@jax.jit
def kernel(input, A, w):
    prod = _tc_matmul(input, A)
    wt = jnp.transpose(w[0], (1, 0))
    return _sc_sort_dot(prod, wt)
